# baseline (speedup 1.0000x reference)
import functools

import jax
import jax.numpy as jnp
from jax.experimental import pallas as pl
from jax.experimental.pallas import tpu as pltpu

_THRESHOLD = 0.7
_TEMPERATURE = 1.0
_BM = 256


def _fused(x_ref, f_ref, w1_ref, b1_ref, ws1_ref, bs1_ref, ws2_ref, bs2_ref,
           out_ref):
    logits = jnp.dot(x_ref[...], w1_ref[...],
                     preferred_element_type=jnp.float32) + b1_ref[...]
    scaled = logits * (1.0 / _TEMPERATURE)
    m = jnp.max(scaled, axis=-1, keepdims=True)
    lse = m + jnp.log(jnp.sum(jnp.exp(scaled - m), axis=-1, keepdims=True))
    fallback = (m - lse) < jnp.log(_THRESHOLD)
    h = jnp.dot(f_ref[...], ws1_ref[...],
                preferred_element_type=jnp.float32) + bs1_ref[...]
    h = jnp.maximum(h, 0.0).astype(jnp.bfloat16)
    sec = jnp.dot(h, ws2_ref[...],
                  preferred_element_type=jnp.float32) + bs2_ref[...]
    out_ref[...] = jnp.where(fallback, sec, logits)


@jax.jit
def kernel(primary_features, fallback_input, W1, b1, Ws1, bs1, Ws2, bs2):
    B, D = primary_features.shape
    H = Ws1.shape[1]
    C = W1.shape[1]
    CP = (C + 127) // 128 * 128

    xb = primary_features.astype(jnp.bfloat16)
    fb = fallback_input.astype(jnp.bfloat16)
    w1 = jnp.pad(W1.astype(jnp.bfloat16), ((0, 0), (0, CP - C)))
    ws1 = Ws1.astype(jnp.bfloat16)
    ws2 = jnp.pad(Ws2.astype(jnp.bfloat16), ((0, 0), (0, CP - C)))
    b1p = jnp.pad(b1.reshape(1, C), ((0, 0), (0, CP - C)),
                  constant_values=-1e30)
    bs1p = bs1.reshape(1, H)
    bs2p = jnp.pad(bs2.reshape(1, C), ((0, 0), (0, CP - C)))

    grid = (B // _BM,)
    row = lambda i: (i, 0)
    const = lambda i: (0, 0)
    out = pl.pallas_call(
        _fused,
        grid=grid,
        in_specs=[
            pl.BlockSpec((_BM, D), row),
            pl.BlockSpec((_BM, D), row),
            pl.BlockSpec((D, CP), const),
            pl.BlockSpec((1, CP), const),
            pl.BlockSpec((D, H), const),
            pl.BlockSpec((1, H), const),
            pl.BlockSpec((H, CP), const),
            pl.BlockSpec((1, CP), const),
        ],
        out_specs=pl.BlockSpec((_BM, CP), row),
        out_shape=jax.ShapeDtypeStruct((B, CP), jnp.float32),
        compiler_params=pltpu.CompilerParams(
            dimension_semantics=("arbitrary",),
        ),
    )(xb, fb, w1, b1p, ws1, bs1p, ws2, bs2p)
    return out[:, :C]


# speedup vs baseline: 1.0007x; 1.0007x over previous
import functools

import jax
import jax.numpy as jnp
from jax.experimental import pallas as pl
from jax.experimental.pallas import tpu as pltpu

_THRESHOLD = 0.7
_TEMPERATURE = 1.0
_BM = 256


def _fused(x_ref, f_ref, w1_ref, b1_ref, ws1_ref, bs1_ref, ws2_ref, bs2_ref,
           out_ref):
    logits = jnp.dot(x_ref[...], w1_ref[...],
                     preferred_element_type=jnp.float32) + b1_ref[...]
    scaled = logits * (1.0 / _TEMPERATURE)
    m = jnp.max(scaled, axis=-1, keepdims=True)
    lse = m + jnp.log(jnp.sum(jnp.exp(scaled - m), axis=-1, keepdims=True))
    fallback = (m - lse) < jnp.log(_THRESHOLD)
    h = jnp.dot(f_ref[...], ws1_ref[...],
                preferred_element_type=jnp.float32) + bs1_ref[...]
    h = jnp.maximum(h, 0.0).astype(jnp.bfloat16)
    sec = jnp.dot(h, ws2_ref[...],
                  preferred_element_type=jnp.float32) + bs2_ref[...]
    out_ref[...] = jnp.where(fallback, sec, logits)


@jax.jit
def kernel(primary_features, fallback_input, W1, b1, Ws1, bs1, Ws2, bs2):
    B, D = primary_features.shape
    H = Ws1.shape[1]
    C = W1.shape[1]
    CP = (C + 127) // 128 * 128

    xb = primary_features.astype(jnp.bfloat16)
    fb = fallback_input.astype(jnp.bfloat16)
    w1 = jnp.pad(W1.astype(jnp.bfloat16), ((0, 0), (0, CP - C)))
    ws1 = Ws1.astype(jnp.bfloat16)
    ws2 = jnp.pad(Ws2.astype(jnp.bfloat16), ((0, 0), (0, CP - C)))
    b1p = jnp.pad(b1.reshape(1, C), ((0, 0), (0, CP - C)),
                  constant_values=-1e30)
    bs1p = bs1.reshape(1, H)
    bs2p = jnp.pad(bs2.reshape(1, C), ((0, 0), (0, CP - C)))

    grid = (B // _BM,)
    row = lambda i: (i, 0)
    const = lambda i: (0, 0)
    out = pl.pallas_call(
        _fused,
        grid=grid,
        in_specs=[
            pl.BlockSpec((_BM, D), row),
            pl.BlockSpec((_BM, D), row),
            pl.BlockSpec((D, CP), const),
            pl.BlockSpec((1, CP), const),
            pl.BlockSpec((D, H), const),
            pl.BlockSpec((1, H), const),
            pl.BlockSpec((H, CP), const),
            pl.BlockSpec((1, CP), const),
        ],
        out_specs=pl.BlockSpec((_BM, CP), row),
        out_shape=jax.ShapeDtypeStruct((B, CP), jnp.float32),
        compiler_params=pltpu.CompilerParams(
            dimension_semantics=("parallel",),
        ),
    )(xb, fb, w1, b1p, ws1, bs1p, ws2, bs2p)
    return out[:, :C]


# speedup vs baseline: 1.0074x; 1.0067x over previous
import functools

import jax
import jax.numpy as jnp
from jax.experimental import pallas as pl
from jax.experimental.pallas import tpu as pltpu

_THRESHOLD = 0.7
_TEMPERATURE = 1.0
_BM = 512


def _fused(x_ref, f_ref, w1_ref, b1_ref, ws1_ref, bs1_ref, ws2_ref, bs2_ref,
           out_ref):
    logits = jnp.dot(x_ref[...], w1_ref[...],
                     preferred_element_type=jnp.float32) + b1_ref[...]
    scaled = logits * (1.0 / _TEMPERATURE)
    m = jnp.max(scaled, axis=-1, keepdims=True)
    lse = m + jnp.log(jnp.sum(jnp.exp(scaled - m), axis=-1, keepdims=True))
    fallback = (m - lse) < jnp.log(_THRESHOLD)
    h = jnp.dot(f_ref[...], ws1_ref[...],
                preferred_element_type=jnp.float32) + bs1_ref[...]
    h = jnp.maximum(h, 0.0).astype(jnp.bfloat16)
    sec = jnp.dot(h, ws2_ref[...],
                  preferred_element_type=jnp.float32) + bs2_ref[...]
    out_ref[...] = jnp.where(fallback, sec, logits)


@jax.jit
def kernel(primary_features, fallback_input, W1, b1, Ws1, bs1, Ws2, bs2):
    B, D = primary_features.shape
    H = Ws1.shape[1]
    C = W1.shape[1]
    CP = (C + 127) // 128 * 128

    xb = primary_features.astype(jnp.bfloat16)
    fb = fallback_input.astype(jnp.bfloat16)
    w1 = jnp.pad(W1.astype(jnp.bfloat16), ((0, 0), (0, CP - C)))
    ws1 = Ws1.astype(jnp.bfloat16)
    ws2 = jnp.pad(Ws2.astype(jnp.bfloat16), ((0, 0), (0, CP - C)))
    b1p = jnp.pad(b1.reshape(1, C), ((0, 0), (0, CP - C)),
                  constant_values=-1e30)
    bs1p = bs1.reshape(1, H)
    bs2p = jnp.pad(bs2.reshape(1, C), ((0, 0), (0, CP - C)))

    grid = (B // _BM,)
    row = lambda i: (i, 0)
    const = lambda i: (0, 0)
    out = pl.pallas_call(
        _fused,
        grid=grid,
        in_specs=[
            pl.BlockSpec((_BM, D), row),
            pl.BlockSpec((_BM, D), row),
            pl.BlockSpec((D, CP), const),
            pl.BlockSpec((1, CP), const),
            pl.BlockSpec((D, H), const),
            pl.BlockSpec((1, H), const),
            pl.BlockSpec((H, CP), const),
            pl.BlockSpec((1, CP), const),
        ],
        out_specs=pl.BlockSpec((_BM, CP), row),
        out_shape=jax.ShapeDtypeStruct((B, CP), jnp.float32),
        compiler_params=pltpu.CompilerParams(
            dimension_semantics=("parallel",),
        ),
    )(xb, fb, w1, b1p, ws1, bs1p, ws2, bs2p)
    return out[:, :C]


# speedup vs baseline: 1.2477x; 1.2385x over previous
import functools

import jax
import jax.numpy as jnp
from jax.experimental import pallas as pl
from jax.experimental.pallas import tpu as pltpu

_THRESHOLD = 0.7
_TEMPERATURE = 1.0
_BM = 256


def _fused(x_ref, f_ref, w1_ref, b1_ref, ws1_ref, bs1_ref, ws2_ref, bs2_ref,
           out_ref, *, n_classes):
    logits = jnp.dot(x_ref[...], w1_ref[...],
                     preferred_element_type=jnp.float32) + b1_ref[...]
    scaled = logits * (1.0 / _TEMPERATURE)
    col = jax.lax.broadcasted_iota(jnp.int32, scaled.shape, 1)
    valid = col < n_classes
    neg = jnp.float32(-jnp.inf)
    m = jnp.max(jnp.where(valid, scaled, neg), axis=-1, keepdims=True)
    s = jnp.sum(jnp.where(valid, jnp.exp(scaled - m), 0.0), axis=-1,
                keepdims=True)
    fallback = (m - (m + jnp.log(s))) < jnp.log(_THRESHOLD)
    h = jnp.dot(f_ref[...], ws1_ref[...],
                preferred_element_type=jnp.float32) + bs1_ref[...]
    h = jnp.maximum(h, 0.0)
    sec = jnp.dot(h, ws2_ref[...],
                  preferred_element_type=jnp.float32) + bs2_ref[...]
    out_ref[...] = jnp.where(fallback, sec, logits)


@jax.jit
def kernel(primary_features, fallback_input, W1, b1, Ws1, bs1, Ws2, bs2):
    B, D = primary_features.shape
    H = Ws1.shape[1]
    C = W1.shape[1]

    grid = (B // _BM,)
    row = lambda i: (i, 0)
    const = lambda i: (0, 0)
    out = pl.pallas_call(
        functools.partial(_fused, n_classes=C),
        grid=grid,
        in_specs=[
            pl.BlockSpec((_BM, D), row),
            pl.BlockSpec((_BM, D), row),
            pl.BlockSpec((D, C), const),
            pl.BlockSpec((1, C), const),
            pl.BlockSpec((D, H), const),
            pl.BlockSpec((1, H), const),
            pl.BlockSpec((H, C), const),
            pl.BlockSpec((1, C), const),
        ],
        out_specs=pl.BlockSpec((_BM, C), row),
        out_shape=jax.ShapeDtypeStruct((B, C), jnp.float32),
        compiler_params=pltpu.CompilerParams(
            dimension_semantics=("arbitrary",),
            vmem_limit_bytes=128 * 1024 * 1024,
        ),
    )(primary_features, fallback_input, W1, b1.reshape(1, C),
      Ws1.astype(jnp.bfloat16), bs1.reshape(1, H), Ws2, bs2.reshape(1, C))
    return out


# speedup vs baseline: 1.3109x; 1.0506x over previous
import functools

import jax
import jax.numpy as jnp
from jax.experimental import pallas as pl
from jax.experimental.pallas import tpu as pltpu

_THRESHOLD = 0.7
_TEMPERATURE = 1.0
_BM = 256


def _fused(x_ref, f_ref, w1_ref, b1_ref, ws1_ref, bs1_ref, ws2_ref, bs2_ref,
           out_ref, *, n_classes):
    logits = jnp.dot(x_ref[...], w1_ref[...],
                     preferred_element_type=jnp.float32) + b1_ref[...]
    scaled = logits * (1.0 / _TEMPERATURE)
    col = jax.lax.broadcasted_iota(jnp.int32, scaled.shape, 1)
    valid = col < n_classes
    neg = jnp.float32(-jnp.inf)
    m = jnp.max(jnp.where(valid, scaled, neg), axis=-1, keepdims=True)
    s = jnp.sum(jnp.where(valid, jnp.exp(scaled - m), 0.0), axis=-1,
                keepdims=True)
    fallback = (m - (m + jnp.log(s))) < jnp.log(_THRESHOLD)
    h = jnp.dot(f_ref[...], ws1_ref[...],
                preferred_element_type=jnp.float32) + bs1_ref[...]
    h = jnp.maximum(h, 0.0)
    sec = jnp.dot(h, ws2_ref[...],
                  preferred_element_type=jnp.float32) + bs2_ref[...]
    out_ref[...] = jnp.where(fallback, sec, logits)


@jax.jit
def kernel(primary_features, fallback_input, W1, b1, Ws1, bs1, Ws2, bs2):
    B, D = primary_features.shape
    H = Ws1.shape[1]
    C = W1.shape[1]

    grid = (B // _BM,)
    row = lambda i: (i, 0)
    const = lambda i: (0, 0)
    out = pl.pallas_call(
        functools.partial(_fused, n_classes=C),
        grid=grid,
        in_specs=[
            pl.BlockSpec((_BM, D), row),
            pl.BlockSpec((_BM, D), row),
            pl.BlockSpec((D, C), const),
            pl.BlockSpec((1, C), const),
            pl.BlockSpec((D, H), const),
            pl.BlockSpec((1, H), const),
            pl.BlockSpec((H, C), const),
            pl.BlockSpec((1, C), const),
        ],
        out_specs=pl.BlockSpec((_BM, C), row),
        out_shape=jax.ShapeDtypeStruct((B, C), jnp.float32),
        compiler_params=pltpu.CompilerParams(
            dimension_semantics=("arbitrary",),
            vmem_limit_bytes=128 * 1024 * 1024,
        ),
    )(primary_features, fallback_input, W1, b1.reshape(1, C),
      Ws1, bs1.reshape(1, H), Ws2.astype(jnp.bfloat16), bs2.reshape(1, C))
    return out
